# Initial kernel scaffold; baseline (speedup 1.0000x reference)
import jax
import jax.numpy as jnp
from jax.experimental import pallas as pl


def kernel(pos, w1a, b1a, w1b, b1b, w2a, b2a, w2b, b2b, w3a, b3a, w3b, b3b):
    raise NotImplementedError("write your pallas kernel here")


# speedup vs baseline: 1.1363x; 1.1363x over previous
import functools

import jax
import jax.numpy as jnp
from jax.experimental import pallas as pl

N = 10000
K = 16


def _knn(pos):
    n = pos.shape[0]
    sq = jnp.sum(pos * pos, axis=1)
    nbrs = []
    chunk = 2000
    for s in range(0, n, chunk):
        p = pos[s:s + chunk]
        d = jnp.sum(p * p, axis=1)[:, None] - 2.0 * (p @ pos.T) + sq[None, :]
        _, idx = jax.lax.top_k(-d, K)
        nbrs.append(idx)
    return jnp.concatenate(nbrs, axis=0)


def _conv_body(nk, g_ref, t_ref, wb_ref, bb_ref, o_ref):
    t = t_ref[...]
    wb = wb_ref[...]
    acc = None
    for k in range(nk):
        pre = jnp.maximum(g_ref[k] - t, 0.0)
        h = jax.lax.dot_general(pre, wb, (((1,), (0,)), ((), ())),
                                preferred_element_type=jnp.float32)
        acc = h if acc is None else jnp.maximum(acc, h)
    o_ref[...] = jnp.maximum(acc + bb_ref[...], 0.0)


def _conv(x, pos, nbr_t, wa, ba, wb, bb, block=1000):
    fin = x.shape[1]
    fo = wb.shape[1]
    s = x @ wa[:fin] + pos @ wa[fin:] + ba
    t = pos @ wa[fin:]
    g = jnp.take(s, nbr_t, axis=0)
    fmid = s.shape[1]
    grid = N // block
    return pl.pallas_call(
        functools.partial(_conv_body, K),
        grid=(grid,),
        in_specs=[
            pl.BlockSpec((K, block, fmid), lambda i: (0, i, 0)),
            pl.BlockSpec((block, fmid), lambda i: (i, 0)),
            pl.BlockSpec((fmid, fo), lambda i: (0, 0)),
            pl.BlockSpec((1, fo), lambda i: (0, 0)),
        ],
        out_specs=pl.BlockSpec((block, fo), lambda i: (i, 0)),
        out_shape=jax.ShapeDtypeStruct((N, fo), jnp.float32),
    )(g, t, wb, bb.reshape(1, fo))


def kernel(pos, w1a, b1a, w1b, b1b, w2a, b2a, w2b, b2b, w3a, b3a, w3b, b3b):
    nbr_t = _knn(pos).T
    h1 = _conv(pos, pos, nbr_t, w1a, b1a, w1b, b1b)
    h2 = _conv(h1, pos, nbr_t, w2a, b2a, w2b, b2b)
    h3 = _conv(h2, pos, nbr_t, w3a, b3a, w3b, b3b)
    return (h1, h2, h3)


# speedup vs baseline: 2.8459x; 2.5045x over previous
import functools

import jax
import jax.numpy as jnp
from jax.experimental import pallas as pl

N = 10000
K = 16
NPAD = 10240


def _knn_body(nk, npad, p_ref, pt_ref, sq_ref, o_ref):
    p = p_ref[...]
    sqp = jnp.sum(p * p, axis=1, keepdims=True)
    d = sqp - 2.0 * jax.lax.dot_general(
        p, pt_ref[...], (((1,), (0,)), ((), ())),
        preferred_element_type=jnp.float32) + sq_ref[...]
    r = p.shape[0]
    colid = jax.lax.broadcasted_iota(jnp.int32, (r, npad), 1)
    kid = jax.lax.broadcasted_iota(jnp.int32, (r, nk), 1)

    def step(t, carry):
        dcur, nbrs = carry
        m = jnp.min(dcur, axis=1, keepdims=True)
        cand = jnp.where(dcur == m, colid, jnp.int32(2**30))
        c = jnp.min(cand, axis=1, keepdims=True)
        nbrs = jnp.where(kid == t, c, nbrs)
        dcur = jnp.where(colid == c, jnp.float32(jnp.inf), dcur)
        return dcur, nbrs

    _, nbrs = jax.lax.fori_loop(
        0, nk, step, (d, jnp.zeros((r, nk), jnp.int32)))
    o_ref[...] = nbrs


def _knn(pos, row_block=256):
    padv = jnp.full((NPAD - N, 3), 1e15, jnp.float32)
    posp = jnp.concatenate([pos, padv], axis=0)
    sq = jnp.sum(posp * posp, axis=1).reshape(1, NPAD)
    grid = NPAD // row_block
    nbr = pl.pallas_call(
        functools.partial(_knn_body, K, NPAD),
        grid=(grid,),
        in_specs=[
            pl.BlockSpec((row_block, 3), lambda i: (i, 0)),
            pl.BlockSpec((3, NPAD), lambda i: (0, 0)),
            pl.BlockSpec((1, NPAD), lambda i: (0, 0)),
        ],
        out_specs=pl.BlockSpec((row_block, K), lambda i: (i, 0)),
        out_shape=jax.ShapeDtypeStruct((NPAD, K), jnp.int32),
    )(posp, posp.T, sq)
    return nbr[:N]


def _conv_body(nk, g_ref, t_ref, wb_ref, bb_ref, o_ref):
    t = t_ref[...]
    wb = wb_ref[...]
    acc = None
    for k in range(nk):
        pre = jnp.maximum(g_ref[k] - t, 0.0)
        h = jax.lax.dot_general(pre, wb, (((1,), (0,)), ((), ())),
                                preferred_element_type=jnp.float32)
        acc = h if acc is None else jnp.maximum(acc, h)
    o_ref[...] = jnp.maximum(acc + bb_ref[...], 0.0)


def _conv(x, pos, nbr_t, wa, ba, wb, bb, block=1000):
    fin = x.shape[1]
    fo = wb.shape[1]
    s = x @ wa[:fin] + pos @ wa[fin:] + ba
    t = pos @ wa[fin:]
    g = jnp.take(s, nbr_t, axis=0)
    fmid = s.shape[1]
    grid = N // block
    return pl.pallas_call(
        functools.partial(_conv_body, K),
        grid=(grid,),
        in_specs=[
            pl.BlockSpec((K, block, fmid), lambda i: (0, i, 0)),
            pl.BlockSpec((block, fmid), lambda i: (i, 0)),
            pl.BlockSpec((fmid, fo), lambda i: (0, 0)),
            pl.BlockSpec((1, fo), lambda i: (0, 0)),
        ],
        out_specs=pl.BlockSpec((block, fo), lambda i: (i, 0)),
        out_shape=jax.ShapeDtypeStruct((N, fo), jnp.float32),
    )(g, t, wb, bb.reshape(1, fo))


def kernel(pos, w1a, b1a, w1b, b1b, w2a, b2a, w2b, b2b, w3a, b3a, w3b, b3b):
    nbr_t = _knn(pos).T
    h1 = _conv(pos, pos, nbr_t, w1a, b1a, w1b, b1b)
    h2 = _conv(h1, pos, nbr_t, w2a, b2a, w2b, b2b)
    h3 = _conv(h2, pos, nbr_t, w3a, b3a, w3b, b3b)
    return (h1, h2, h3)


# speedup vs baseline: 3.5810x; 1.2583x over previous
import functools

import jax
import jax.numpy as jnp
from jax import lax
from jax.experimental import pallas as pl
from jax.experimental.pallas import tpu as pltpu
from jax.experimental.pallas import tpu_sc as plsc

N = 10000
K = 16
NPAD = 10240
_NC = 2
_NS = 16


def _knn_body(nk, npad, p_ref, pt_ref, sq_ref, o_ref):
    p = p_ref[...]
    sqp = jnp.sum(p * p, axis=1, keepdims=True)
    d = sqp - 2.0 * jax.lax.dot_general(
        p, pt_ref[...], (((1,), (0,)), ((), ())),
        preferred_element_type=jnp.float32) + sq_ref[...]
    r = p.shape[0]
    colid = jax.lax.broadcasted_iota(jnp.int32, (r, npad), 1)
    kid = jax.lax.broadcasted_iota(jnp.int32, (r, nk), 1)

    def step(t, carry):
        dcur, nbrs = carry
        m = jnp.min(dcur, axis=1, keepdims=True)
        cand = jnp.where(dcur == m, colid, jnp.int32(2**30))
        c = jnp.min(cand, axis=1, keepdims=True)
        nbrs = jnp.where(kid == t, c, nbrs)
        dcur = jnp.where(colid == c, jnp.float32(jnp.inf), dcur)
        return dcur, nbrs

    _, nbrs = jax.lax.fori_loop(
        0, nk, step, (d, jnp.zeros((r, nk), jnp.int32)))
    o_ref[...] = nbrs


def _knn(pos, row_block=256):
    padv = jnp.full((NPAD - N, 3), 1e15, jnp.float32)
    posp = jnp.concatenate([pos, padv], axis=0)
    sq = jnp.sum(posp * posp, axis=1).reshape(1, NPAD)
    grid = NPAD // row_block
    nbr = pl.pallas_call(
        functools.partial(_knn_body, K, NPAD),
        grid=(grid,),
        in_specs=[
            pl.BlockSpec((row_block, 3), lambda i: (i, 0)),
            pl.BlockSpec((3, NPAD), lambda i: (0, 0)),
            pl.BlockSpec((1, NPAD), lambda i: (0, 0)),
        ],
        out_specs=pl.BlockSpec((row_block, K), lambda i: (i, 0)),
        out_shape=jax.ShapeDtypeStruct((NPAD, K), jnp.int32),
    )(posp, posp.T, sq)
    return nbr[:N]


def _sc_gather(table, idx_flat):
    b = idx_flat.shape[0]
    f = table.shape[1]
    nw = _NC * _NS
    b_per_w = b // nw
    chunk = 200
    nchunks = b_per_w // chunk
    mesh = plsc.VectorSubcoreMesh(core_axis_name="c", subcore_axis_name="s",
                                  num_cores=_NC, num_subcores=_NS)

    @functools.partial(
        pl.kernel, mesh=mesh,
        out_type=jax.ShapeDtypeStruct((b, f), jnp.float32),
        scratch_types=[
            pltpu.VMEM((chunk,), jnp.int32),
            pltpu.VMEM((chunk, f), jnp.float32),
            pltpu.SemaphoreType.DMA,
        ])
    def gather_k(table_hbm, idx_hbm, out_hbm, idx_v, rows_v, sem):
        wid = lax.axis_index("s") * _NC + lax.axis_index("c")
        base = wid * b_per_w

        def body(c, carry):
            off = base + c * chunk
            pltpu.sync_copy(idx_hbm.at[pl.ds(off, chunk)], idx_v)
            pltpu.async_copy(table_hbm.at[idx_v], rows_v, sem).wait()
            pltpu.sync_copy(rows_v, out_hbm.at[pl.ds(off, chunk)])
            return carry

        lax.fori_loop(0, nchunks, body, 0)

    return gather_k(table, idx_flat)


def _conv_body(nk, g_ref, t_ref, wb_ref, bb_ref, o_ref):
    t = t_ref[...]
    wb = wb_ref[...]
    acc = None
    for k in range(nk):
        pre = jnp.maximum(g_ref[k] - t, 0.0)
        h = jax.lax.dot_general(pre, wb, (((1,), (0,)), ((), ())),
                                preferred_element_type=jnp.float32)
        acc = h if acc is None else jnp.maximum(acc, h)
    o_ref[...] = jnp.maximum(acc + bb_ref[...], 0.0)


def _conv(x, pos, nbr_t, wa, ba, wb, bb, block=1000):
    fin = x.shape[1]
    fo = wb.shape[1]
    s = x @ wa[:fin] + pos @ wa[fin:] + ba
    t = pos @ wa[fin:]
    fmid = 128
    pad = fmid - s.shape[1]
    if pad:
        s = jnp.pad(s, ((0, 0), (0, pad)))
        t = jnp.pad(t, ((0, 0), (0, pad)))
        wb = jnp.pad(wb, ((0, pad), (0, 0)))
    g = _sc_gather(s, nbr_t.reshape(-1)).reshape(K, N, fmid)
    grid = N // block
    return pl.pallas_call(
        functools.partial(_conv_body, K),
        grid=(grid,),
        in_specs=[
            pl.BlockSpec((K, block, fmid), lambda i: (0, i, 0)),
            pl.BlockSpec((block, fmid), lambda i: (i, 0)),
            pl.BlockSpec((fmid, fo), lambda i: (0, 0)),
            pl.BlockSpec((1, fo), lambda i: (0, 0)),
        ],
        out_specs=pl.BlockSpec((block, fo), lambda i: (i, 0)),
        out_shape=jax.ShapeDtypeStruct((N, fo), jnp.float32),
    )(g, t, wb, bb.reshape(1, fo))


def kernel(pos, w1a, b1a, w1b, b1b, w2a, b2a, w2b, b2b, w3a, b3a, w3b, b3b):
    nbr_t = _knn(pos).T
    h1 = _conv(pos, pos, nbr_t, w1a, b1a, w1b, b1b)
    h2 = _conv(h1, pos, nbr_t, w2a, b2a, w2b, b2b)
    h3 = _conv(h2, pos, nbr_t, w3a, b3a, w3b, b3b)
    return (h1, h2, h3)


# speedup vs baseline: 9.3961x; 2.6239x over previous
import functools

import jax
import jax.numpy as jnp
from jax import lax
from jax.experimental import pallas as pl
from jax.experimental.pallas import tpu as pltpu
from jax.experimental.pallas import tpu_sc as plsc

N = 10000
K = 16
NPAD = 10240
_NC = 2
_NS = 16


_SG = 16
_NSG = NPAD // _SG
_H = 32
_CW = _H * _SG


def _knn_a_body(npad, p_ref, pt_ref, ptb_ref, pall_ref, sq_ref, sqc_ref,
                d_ref, gmt_ref):
    p = p_ref[...]
    sqp = jnp.sum(p * p, axis=1, keepdims=True)
    d = sqp - 2.0 * jax.lax.dot_general(
        p, pt_ref[...], (((1,), (0,)), ((), ())),
        preferred_element_type=jnp.float32) + sq_ref[...]
    d_ref[...] = d
    r = p.shape[0]
    ptb = ptb_ref[...]
    sqp_row = jnp.sum(ptb * ptb, axis=0, keepdims=True)
    dt = sqc_ref[...] - 2.0 * jax.lax.dot_general(
        pall_ref[...], ptb, (((1,), (0,)), ((), ())),
        preferred_element_type=jnp.float32) + sqp_row
    gmt_ref[...] = jnp.min(dt.reshape(_NSG, _SG, r), axis=1)


def _hits_body(nh, g_ref, o_ref):
    g = g_ref[...]
    r = g.shape[0]
    colid = jax.lax.broadcasted_iota(jnp.int32, (r, _NSG), 1)
    sid = jax.lax.broadcasted_iota(jnp.int32, (r, 128), 1)
    bits = jax.lax.bitcast_convert_type(g, jnp.int32)
    srt = jnp.where(bits >= 0, bits, bits ^ jnp.int32(0x7FFFFFFF))
    key = (srt & jnp.int32(-16384)) | colid

    def step(t, carry):
        w, hits = carry
        m = jnp.min(w, axis=1, keepdims=True)
        hits = jnp.where(sid == t, m & jnp.int32(0x3FF), hits)
        w = jnp.where(w == m, jnp.int32(2**31 - 1), w)
        return w, hits

    _, hits = jax.lax.fori_loop(
        0, nh, step, (key, jnp.zeros((r, 128), jnp.int32)))
    o_ref[...] = hits


def _sc_compact(d, hits):
    rows_pw = NPAD // (_NC * _NS)
    mesh = plsc.VectorSubcoreMesh(core_axis_name="c", subcore_axis_name="s",
                                  num_cores=_NC, num_subcores=_NS)
    @functools.partial(
        pl.kernel, mesh=mesh,
        out_type=(jax.ShapeDtypeStruct((NPAD, _CW), jnp.float32),
                  jax.ShapeDtypeStruct((NPAD, _CW), jnp.int32)),
        scratch_types=[
            pltpu.VMEM((2, NPAD), jnp.float32),
            pltpu.VMEM((rows_pw, 128), jnp.int32),
            pltpu.VMEM((_CW,), jnp.float32),
            pltpu.VMEM((_CW,), jnp.float32),
            pltpu.VMEM((_CW,), jnp.int32),
            pltpu.VMEM((_CW,), jnp.int32),
            pltpu.SemaphoreType.DMA,
            pltpu.SemaphoreType.DMA,
            pltpu.SemaphoreType.DMA,
            pltpu.SemaphoreType.DMA,
        ])
    def compact_k(d_hbm, hits_hbm, vals_hbm, idxs_hbm, drow, hitsb,
                  ov0, ov1, oi0, oi1, sd0, sd1, so0, so1):
        wid = lax.axis_index("s") * _NC + lax.axis_index("c")
        r0 = wid * rows_pw
        lane = jax.lax.iota(jnp.int32, 16)
        pltpu.sync_copy(hits_hbm.at[pl.ds(r0, rows_pw)], hitsb)
        pltpu.async_copy(d_hbm.at[r0], drow.at[0], sd0)
        pltpu.async_copy(d_hbm.at[r0 + 1], drow.at[1], sd1)

        def pair_body(ii, carry):
            for b in range(2):
                ov = ov0 if b == 0 else ov1
                oi = oi0 if b == 0 else oi1
                sd = sd0 if b == 0 else sd1
                so = so0 if b == 0 else so1
                r = ii * 2 + b
                pltpu.make_async_copy(d_hbm.at[r0], drow.at[b], sd).wait()

                @pl.when(ii >= 1)
                def _drain():
                    pltpu.make_async_copy(ov, vals_hbm.at[r0], so).wait()
                    pltpu.make_async_copy(oi, idxs_hbm.at[r0], so).wait()

                for hc in range(_H // 16):
                    hv = hitsb[r, pl.ds(hc * 16, 16)]
                    for u in range(16):
                        slot = hc * 16 + u
                        off = hv[u] * _SG
                        ov[pl.ds(slot * 16, 16)] = drow[b, pl.ds(off, 16)]
                        oi[pl.ds(slot * 16, 16)] = lane + off

                @pl.when(r + 2 < rows_pw)
                def _prefetch():
                    pltpu.async_copy(d_hbm.at[r0 + r + 2], drow.at[b], sd)

                pltpu.async_copy(ov, vals_hbm.at[r0 + r], so)
                pltpu.async_copy(oi, idxs_hbm.at[r0 + r], so)
            return carry

        lax.fori_loop(0, rows_pw // 2, pair_body, 0)
        for b in range(2):
            ov = ov0 if b == 0 else ov1
            oi = oi0 if b == 0 else oi1
            so = so0 if b == 0 else so1
            pltpu.make_async_copy(ov, vals_hbm.at[r0], so).wait()
            pltpu.make_async_copy(oi, idxs_hbm.at[r0], so).wait()

    return compact_k(d, hits)


def _knn_c_body(nk, v_ref, i_ref, o_ref):
    v = v_ref[...]
    ix = i_ref[...]
    r = v.shape[0]
    kid = jax.lax.broadcasted_iota(jnp.int32, (r, nk), 1)

    def step(t, carry):
        vcur, nbrs = carry
        m = jnp.min(vcur, axis=1, keepdims=True)
        c = jnp.min(jnp.where(vcur == m, ix, jnp.int32(2**30)),
                    axis=1, keepdims=True)
        nbrs = jnp.where(kid == t, c, nbrs)
        vcur = jnp.where(ix == c, jnp.float32(jnp.inf), vcur)
        return vcur, nbrs

    _, nbrs = jax.lax.fori_loop(
        0, nk, step, (v, jnp.zeros((r, nk), jnp.int32)))
    o_ref[...] = nbrs


def _knn(pos, row_block=256):
    padv = jnp.full((NPAD - N, 3), 1e15, jnp.float32)
    posp = jnp.concatenate([pos, padv], axis=0)
    sq = jnp.sum(posp * posp, axis=1)
    grid = NPAD // row_block
    d, gmt = pl.pallas_call(
        functools.partial(_knn_a_body, NPAD),
        grid=(grid,),
        in_specs=[
            pl.BlockSpec((row_block, 3), lambda i: (i, 0)),
            pl.BlockSpec((3, NPAD), lambda i: (0, 0)),
            pl.BlockSpec((3, row_block), lambda i: (0, i)),
            pl.BlockSpec((NPAD, 3), lambda i: (0, 0)),
            pl.BlockSpec((1, NPAD), lambda i: (0, 0)),
            pl.BlockSpec((NPAD, 1), lambda i: (0, 0)),
        ],
        out_specs=[
            pl.BlockSpec((row_block, NPAD), lambda i: (i, 0)),
            pl.BlockSpec((_NSG, row_block), lambda i: (0, i)),
        ],
        out_shape=[
            jax.ShapeDtypeStruct((NPAD, NPAD), jnp.float32),
            jax.ShapeDtypeStruct((_NSG, NPAD), jnp.float32),
        ],
    )(posp, posp.T, posp.T, posp, sq.reshape(1, NPAD), sq.reshape(NPAD, 1))
    gm = gmt.T
    hblock = 2048
    hits = pl.pallas_call(
        functools.partial(_hits_body, _H),
        grid=(NPAD // hblock,),
        in_specs=[pl.BlockSpec((hblock, _NSG), lambda i: (i, 0))],
        out_specs=pl.BlockSpec((hblock, 128), lambda i: (i, 0)),
        out_shape=jax.ShapeDtypeStruct((NPAD, 128), jnp.int32),
    )(gm)
    vals, idxs = _sc_compact(d, hits)
    cblock = 1024
    nbr = pl.pallas_call(
        functools.partial(_knn_c_body, K),
        grid=(NPAD // cblock,),
        in_specs=[
            pl.BlockSpec((cblock, _CW), lambda i: (i, 0)),
            pl.BlockSpec((cblock, _CW), lambda i: (i, 0)),
        ],
        out_specs=pl.BlockSpec((cblock, K), lambda i: (i, 0)),
        out_shape=jax.ShapeDtypeStruct((NPAD, K), jnp.int32),
    )(vals, idxs)
    return nbr[:N]


def _sc_gather(table, idx_flat):
    b = idx_flat.shape[0]
    f = table.shape[1]
    nw = _NC * _NS
    b_per_w = b // nw
    chunk = 200
    nchunks = b_per_w // chunk
    mesh = plsc.VectorSubcoreMesh(core_axis_name="c", subcore_axis_name="s",
                                  num_cores=_NC, num_subcores=_NS)

    @functools.partial(
        pl.kernel, mesh=mesh,
        out_type=jax.ShapeDtypeStruct((b, f), jnp.float32),
        scratch_types=[
            pltpu.VMEM((chunk,), jnp.int32),
            pltpu.VMEM((chunk, f), jnp.float32),
            pltpu.SemaphoreType.DMA,
        ])
    def gather_k(table_hbm, idx_hbm, out_hbm, idx_v, rows_v, sem):
        wid = lax.axis_index("s") * _NC + lax.axis_index("c")
        base = wid * b_per_w

        def body(c, carry):
            off = base + c * chunk
            pltpu.sync_copy(idx_hbm.at[pl.ds(off, chunk)], idx_v)
            pltpu.async_copy(table_hbm.at[idx_v], rows_v, sem).wait()
            pltpu.sync_copy(rows_v, out_hbm.at[pl.ds(off, chunk)])
            return carry

        lax.fori_loop(0, nchunks, body, 0)

    return gather_k(table, idx_flat)


def _conv_body(nk, g_ref, t_ref, wb_ref, bb_ref, o_ref):
    t = t_ref[...]
    wb = wb_ref[...]
    acc = None
    for k in range(nk):
        pre = jnp.maximum(g_ref[k] - t, 0.0)
        h = jax.lax.dot_general(pre, wb, (((1,), (0,)), ((), ())),
                                preferred_element_type=jnp.float32)
        acc = h if acc is None else jnp.maximum(acc, h)
    o_ref[...] = jnp.maximum(acc + bb_ref[...], 0.0)


def _conv(x, pos, nbr_t, wa, ba, wb, bb, block=1000):
    fin = x.shape[1]
    fo = wb.shape[1]
    s = x @ wa[:fin] + pos @ wa[fin:] + ba
    t = pos @ wa[fin:]
    fmid = 128
    pad = fmid - s.shape[1]
    if pad:
        s = jnp.pad(s, ((0, 0), (0, pad)))
        t = jnp.pad(t, ((0, 0), (0, pad)))
        wb = jnp.pad(wb, ((0, pad), (0, 0)))
    g = _sc_gather(s, nbr_t.reshape(-1)).reshape(K, N, fmid)
    grid = N // block
    return pl.pallas_call(
        functools.partial(_conv_body, K),
        grid=(grid,),
        in_specs=[
            pl.BlockSpec((K, block, fmid), lambda i: (0, i, 0)),
            pl.BlockSpec((block, fmid), lambda i: (i, 0)),
            pl.BlockSpec((fmid, fo), lambda i: (0, 0)),
            pl.BlockSpec((1, fo), lambda i: (0, 0)),
        ],
        out_specs=pl.BlockSpec((block, fo), lambda i: (i, 0)),
        out_shape=jax.ShapeDtypeStruct((N, fo), jnp.float32),
    )(g, t, wb, bb.reshape(1, fo))


def kernel(pos, w1a, b1a, w1b, b1b, w2a, b2a, w2b, b2b, w3a, b3a, w3b, b3b):
    nbr_t = _knn(pos).T
    h1 = _conv(pos, pos, nbr_t, w1a, b1a, w1b, b1b)
    h2 = _conv(h1, pos, nbr_t, w2a, b2a, w2b, b2b)
    h3 = _conv(h2, pos, nbr_t, w3a, b3a, w3b, b3b)
    return (h1, h2, h3)
